# baseline (speedup 1.0000x reference)
import functools

import jax
import jax.numpy as jnp
from jax import lax
from jax.experimental import pallas as pl
from jax.experimental.pallas import tpu as pltpu


def _agg_kernel(x_ref, o_ref, *, t, inv_t, band0_end, band2_start):
    bblk, _, c = x_ref.shape
    g = c // t
    n = bblk * t
    a0_end = band0_end // g
    a2_start = band2_start // g

    xf = x_ref[...].reshape(n, c)
    xb = xf.astype(jnp.bfloat16)

    nq = 4 * g
    jj = lax.broadcasted_iota(jnp.int32, (c, nq), 0)
    cc = lax.broadcasted_iota(jnp.int32, (c, nq), 1)
    p = cc % g
    blk = cc // g
    qg = (((blk == 0) & (jj // t == p))
          | ((blk == 1) & (jj == p * t))
          | ((blk == 2) & (jj == p * t + t - 1))).astype(jnp.bfloat16)

    y = jnp.dot(xb, qg, preferred_element_type=jnp.float32)
    sums = y[:, 0:g]
    first = y[:, g:2 * g]
    last = y[:, 2 * g:3 * g]

    arow = lax.broadcasted_iota(jnp.int32, (n, g), 0) % t
    conv = (sums
            - jnp.where(arow < a0_end, last, 0.0)
            - jnp.where(arow >= a2_start, first, 0.0))
    conv3 = conv.reshape(bblk, t, g)
    convr = jnp.concatenate([conv3[:, a, :] for a in range(t)], axis=1)

    rb = lax.broadcasted_iota(jnp.int32, (bblk, n), 0)
    rr = lax.broadcasted_iota(jnp.int32, (bblk, n), 1)
    s = (rr // t == rb).astype(jnp.bfloat16)
    res = jnp.dot(s, xb, preferred_element_type=jnp.float32)

    o_ref[...] = ((convr + res) * inv_t).astype(o_ref.dtype)


def kernel(x):
    b, t, c = x.shape
    assert c % t == 0
    g = c // t
    band0_end = c // 4
    band2_start = c + (-c // 4)
    assert band0_end % g == 0 and band2_start % g == 0
    bblk = min(b, 128)
    params = pltpu.CompilerParams(
        dimension_semantics=("parallel",),
        vmem_limit_bytes=52 << 20,
    )
    return pl.pallas_call(
        functools.partial(
            _agg_kernel, t=t, inv_t=1.0 / t,
            band0_end=band0_end, band2_start=band2_start),
        out_shape=jax.ShapeDtypeStruct((b, c), x.dtype),
        grid=(pl.cdiv(b, bblk),),
        in_specs=[pl.BlockSpec((bblk, t, c), lambda i: (i, 0, 0))],
        out_specs=pl.BlockSpec((bblk, c), lambda i: (i, 0)),
        compiler_params=params,
    )(x)


# speedup vs baseline: 1.2996x; 1.2996x over previous
import functools

import jax
import jax.numpy as jnp
from jax import lax
from jax.experimental import pallas as pl
from jax.experimental.pallas import tpu as pltpu


def _agg_kernel(x_ref, o_ref, *, t, inv_t, band0_end, band2_start):
    bblk, _, c = x_ref.shape
    g = c // t
    n = bblk * t
    a0_end = band0_end // g
    a2_start = band2_start // g

    nq = 4 * g
    jj = lax.broadcasted_iota(jnp.int32, (c, nq), 0)
    cc = lax.broadcasted_iota(jnp.int32, (c, nq), 1)
    p = cc % g
    blk = cc // g
    qg = (((blk == 0) & (jj // t == p))
          | ((blk == 1) & (jj == p * t))
          | ((blk == 2) & (jj == p * t + t - 1))).astype(jnp.bfloat16)

    bh = bblk // 2
    nh = bh * t

    rb = lax.broadcasted_iota(jnp.int32, (bh, nh), 0)
    rr = lax.broadcasted_iota(jnp.int32, (bh, nh), 1)
    s = (rr // t == rb).astype(jnp.bfloat16)

    arow = lax.broadcasted_iota(jnp.int32, (nh, g), 0) % t
    sub0 = arow < a0_end
    sub2 = arow >= a2_start

    for h in range(2):
        xf = x_ref[h * bh:(h + 1) * bh].reshape(nh, c)
        xb = xf.astype(jnp.bfloat16)

        y = jnp.dot(xb, qg, preferred_element_type=jnp.float32)
        conv = (y[:, 0:g]
                - jnp.where(sub0, y[:, 2 * g:3 * g], 0.0)
                - jnp.where(sub2, y[:, g:2 * g], 0.0))
        conv3 = conv.reshape(bh, t, g)
        convr = jnp.concatenate([conv3[:, a, :] for a in range(t)], axis=1)

        res = jnp.dot(s, xb, preferred_element_type=jnp.float32)
        o_ref[h * bh:(h + 1) * bh, :] = (
            (convr + res) * inv_t).astype(o_ref.dtype)


def kernel(x):
    b, t, c = x.shape
    assert c % t == 0
    g = c // t
    band0_end = c // 4
    band2_start = c + (-c // 4)
    assert band0_end % g == 0 and band2_start % g == 0
    bblk = min(b, 256)
    params = pltpu.CompilerParams(
        dimension_semantics=("parallel",),
        vmem_limit_bytes=52 << 20,
    )
    return pl.pallas_call(
        functools.partial(
            _agg_kernel, t=t, inv_t=1.0 / t,
            band0_end=band0_end, band2_start=band2_start),
        out_shape=jax.ShapeDtypeStruct((b, c), x.dtype),
        grid=(pl.cdiv(b, bblk),),
        in_specs=[pl.BlockSpec((bblk, t, c), lambda i: (i, 0, 0))],
        out_specs=pl.BlockSpec((bblk, c), lambda i: (i, 0)),
        compiler_params=params,
    )(x)


# speedup vs baseline: 1.3372x; 1.0289x over previous
import functools

import jax
import jax.numpy as jnp
from jax import lax
from jax.experimental import pallas as pl
from jax.experimental.pallas import tpu as pltpu


def _agg_kernel(x_ref, o_ref, *, t, inv_t, band0_end, band2_start):
    bblk, _, c = x_ref.shape
    g = c // t
    n = bblk * t
    a0_end = band0_end // g
    a2_start = band2_start // g

    nq = 4 * g
    jj = lax.broadcasted_iota(jnp.int32, (c, nq), 0)
    cc = lax.broadcasted_iota(jnp.int32, (c, nq), 1)
    p = cc % g
    blk = cc // g
    qg = (((blk == 0) & (jj // t == p))
          | ((blk == 1) & (jj == p * t))
          | ((blk == 2) & (jj == p * t + t - 1))).astype(jnp.bfloat16)

    bh = bblk // 4
    nh = bh * t

    rb = lax.broadcasted_iota(jnp.int32, (bh, nh), 0)
    rr = lax.broadcasted_iota(jnp.int32, (bh, nh), 1)
    s = (rr // t == rb).astype(jnp.bfloat16)

    arow = lax.broadcasted_iota(jnp.int32, (nh, g), 0) % t
    sub0 = arow < a0_end
    sub2 = arow >= a2_start

    for h in range(4):
        xf = x_ref[h * bh:(h + 1) * bh].reshape(nh, c)
        xb = xf.astype(jnp.bfloat16)

        y = jnp.dot(xb, qg, preferred_element_type=jnp.float32)
        conv = (y[:, 0:g]
                - jnp.where(sub0, y[:, 2 * g:3 * g], 0.0)
                - jnp.where(sub2, y[:, g:2 * g], 0.0))
        conv3 = conv.reshape(bh, t, g)
        convr = jnp.concatenate([conv3[:, a, :] for a in range(t)], axis=1)

        res = jnp.dot(s, xb, preferred_element_type=jnp.float32)
        o_ref[h * bh:(h + 1) * bh, :] = (
            (convr + res) * inv_t).astype(o_ref.dtype)


def kernel(x):
    b, t, c = x.shape
    assert c % t == 0
    g = c // t
    band0_end = c // 4
    band2_start = c + (-c // 4)
    assert band0_end % g == 0 and band2_start % g == 0
    bblk = min(b, 256)
    params = pltpu.CompilerParams(
        dimension_semantics=("parallel",),
        vmem_limit_bytes=52 << 20,
    )
    return pl.pallas_call(
        functools.partial(
            _agg_kernel, t=t, inv_t=1.0 / t,
            band0_end=band0_end, band2_start=band2_start),
        out_shape=jax.ShapeDtypeStruct((b, c), x.dtype),
        grid=(pl.cdiv(b, bblk),),
        in_specs=[pl.BlockSpec((bblk, t, c), lambda i: (i, 0, 0))],
        out_specs=pl.BlockSpec((bblk, c), lambda i: (i, 0)),
        compiler_params=params,
    )(x)


# speedup vs baseline: 1.5834x; 1.1842x over previous
import functools

import jax
import jax.numpy as jnp
from jax import lax
from jax.experimental import pallas as pl
from jax.experimental.pallas import tpu as pltpu


def _agg_kernel(x_ref, o_ref, *, t, inv_t, band0_end, band2_start):
    bblk, _, c = x_ref.shape
    g = c // t
    n = bblk * t
    a0_end = band0_end // g
    a2_start = band2_start // g

    nq = 4 * g
    jj = lax.broadcasted_iota(jnp.int32, (c, nq), 0)
    cc = lax.broadcasted_iota(jnp.int32, (c, nq), 1)
    p = cc % g
    blk = cc // g
    qg = (((blk == 0) & (jj // t == p))
          | ((blk == 1) & (jj == p * t))
          | ((blk == 2) & (jj == p * t + t - 1))).astype(jnp.bfloat16)

    bh = bblk // 8
    nh = bh * t

    rb = lax.broadcasted_iota(jnp.int32, (bh, nh), 0)
    rr = lax.broadcasted_iota(jnp.int32, (bh, nh), 1)
    s = (rr // t == rb).astype(jnp.bfloat16)

    arow = lax.broadcasted_iota(jnp.int32, (nh, g), 0) % t
    sub0 = arow < a0_end
    sub2 = arow >= a2_start

    for h in range(8):
        xf = x_ref[h * bh:(h + 1) * bh].reshape(nh, c)
        xb = xf.astype(jnp.bfloat16)

        y = jnp.dot(xb, qg, preferred_element_type=jnp.float32)
        conv = (y[:, 0:g]
                - jnp.where(sub0, y[:, 2 * g:3 * g], 0.0)
                - jnp.where(sub2, y[:, g:2 * g], 0.0))
        conv3 = conv.reshape(bh, t, g)
        convr = jnp.concatenate([conv3[:, a, :] for a in range(t)], axis=1)

        res = jnp.dot(s, xb, preferred_element_type=jnp.float32)
        o_ref[h * bh:(h + 1) * bh, :] = (
            (convr + res) * inv_t).astype(o_ref.dtype)


def kernel(x):
    b, t, c = x.shape
    assert c % t == 0
    g = c // t
    band0_end = c // 4
    band2_start = c + (-c // 4)
    assert band0_end % g == 0 and band2_start % g == 0
    bblk = min(b, 512)
    params = pltpu.CompilerParams(
        dimension_semantics=("parallel",),
        vmem_limit_bytes=52 << 20,
    )
    return pl.pallas_call(
        functools.partial(
            _agg_kernel, t=t, inv_t=1.0 / t,
            band0_end=band0_end, band2_start=band2_start),
        out_shape=jax.ShapeDtypeStruct((b, c), x.dtype),
        grid=(pl.cdiv(b, bblk),),
        in_specs=[pl.BlockSpec((bblk, t, c), lambda i: (i, 0, 0))],
        out_specs=pl.BlockSpec((bblk, c), lambda i: (i, 0)),
        compiler_params=params,
    )(x)


# speedup vs baseline: 1.7037x; 1.0760x over previous
import functools

import jax
import jax.numpy as jnp
from jax import lax
from jax.experimental import pallas as pl
from jax.experimental.pallas import tpu as pltpu


def _agg_kernel(x_ref, o_ref, *, t, inv_t, band0_end, band2_start):
    bblk, _, c = x_ref.shape
    g = c // t
    n = bblk * t
    a0_end = band0_end // g
    a2_start = band2_start // g

    nq = 4 * g
    jj = lax.broadcasted_iota(jnp.int32, (c, nq), 0)
    cc = lax.broadcasted_iota(jnp.int32, (c, nq), 1)
    p = cc % g
    blk = cc // g
    qg = (((blk == 0) & (jj // t == p))
          | ((blk == 1) & (jj == p * t))
          | ((blk == 2) & (jj == p * t + t - 1))).astype(jnp.bfloat16)

    bh = bblk // 16
    nh = bh * t

    rb = lax.broadcasted_iota(jnp.int32, (bh, nh), 0)
    rr = lax.broadcasted_iota(jnp.int32, (bh, nh), 1)
    s = (rr // t == rb).astype(jnp.bfloat16)

    arow = lax.broadcasted_iota(jnp.int32, (nh, g), 0) % t
    sub0 = arow < a0_end
    sub2 = arow >= a2_start

    for h in range(16):
        xf = x_ref[h * bh:(h + 1) * bh].reshape(nh, c)
        xb = xf.astype(jnp.bfloat16)

        y = jnp.dot(xb, qg, preferred_element_type=jnp.float32)
        conv = (y[:, 0:g]
                - jnp.where(sub0, y[:, 2 * g:3 * g], 0.0)
                - jnp.where(sub2, y[:, g:2 * g], 0.0))
        conv3 = conv.reshape(bh, t, g)
        convr = jnp.concatenate([conv3[:, a, :] for a in range(t)], axis=1)

        res = jnp.dot(s, xb, preferred_element_type=jnp.float32)
        o_ref[h * bh:(h + 1) * bh, :] = (
            (convr + res) * inv_t).astype(o_ref.dtype)


def kernel(x):
    b, t, c = x.shape
    assert c % t == 0
    g = c // t
    band0_end = c // 4
    band2_start = c + (-c // 4)
    assert band0_end % g == 0 and band2_start % g == 0
    bblk = min(b, 1024)
    params = pltpu.CompilerParams(
        dimension_semantics=("parallel",),
        vmem_limit_bytes=52 << 20,
    )
    return pl.pallas_call(
        functools.partial(
            _agg_kernel, t=t, inv_t=1.0 / t,
            band0_end=band0_end, band2_start=band2_start),
        out_shape=jax.ShapeDtypeStruct((b, c), x.dtype),
        grid=(pl.cdiv(b, bblk),),
        in_specs=[pl.BlockSpec((bblk, t, c), lambda i: (i, 0, 0))],
        out_specs=pl.BlockSpec((bblk, c), lambda i: (i, 0)),
        compiler_params=params,
    )(x)
